# baseline (speedup 1.0000x reference)
import functools

import jax
import jax.numpy as jnp
from jax import lax
from jax.experimental import pallas as pl
from jax.experimental.pallas import tpu as pltpu
from jax.experimental.pallas import tpu_sc as plsc

_B, _S, _D = 4, 8192, 1024
_NW = 32
_SEQ_PER_W = _S // _NW
_R = 32
_NSC = _SEQ_PER_W // _R
_NSUB = _NSC * _B
_CHUNK = _R * _D

_mesh = plsc.VectorSubcoreMesh(core_axis_name="c", subcore_axis_name="s")


@functools.partial(
    pl.kernel,
    mesh=_mesh,
    out_type=jax.ShapeDtypeStruct((_B * _S * _D,), jnp.float32),
    scratch_types=[
        pltpu.VMEM((_CHUNK,), jnp.float32),
        pltpu.VMEM((_CHUNK,), jnp.float32),
        pltpu.VMEM((_CHUNK,), jnp.float32),
        pltpu.VMEM((_CHUNK,), jnp.float32),
        pltpu.SemaphoreType.DMA,
        pltpu.SemaphoreType.DMA,
        pltpu.SemaphoreType.DMA,
        pltpu.SemaphoreType.DMA,
        pltpu.SemaphoreType.DMA,
        pltpu.SemaphoreType.DMA,
    ],
)
def _sc_add(x_hbm, pe_hbm, out_hbm, xb0, xb1, pb0, pb1,
            ix0, ix1, ip0, ip1, o0, o1):
    wid = lax.axis_index("s") * 2 + lax.axis_index("c")
    seq0 = wid * _SEQ_PER_W
    xbufs = (xb0, xb1)
    pbufs = (pb0, pb1)
    ix_sems = (ix0, ix1)
    ip_sems = (ip0, ip1)
    out_sems = (o0, o1)

    def x_off(j):
        sc, b = j // _B, j % _B
        return (b * _S + seq0 + sc * _R) * _D

    def fetch_x(j):
        slot = j % 2
        return pltpu.async_copy(
            x_hbm.at[pl.ds(x_off(j), _CHUNK)], xbufs[slot], ix_sems[slot])

    def fetch_pe(sc):
        slot = sc % 2
        off = (seq0 + sc * _R) * _D
        return pltpu.async_copy(
            pe_hbm.at[pl.ds(off, _CHUNK)], pbufs[slot], ip_sems[slot])

    def add_loop(xb, pb):
        def add16(i, carry):
            sl = pl.ds(i * 16, 16)
            plsc.addupdate(xb.at[sl], pb[sl])
            return carry
        lax.fori_loop(0, _CHUNK // 16, add16, 0, unroll=16)

    x_fetch = [None] * 2
    pe_fetch = [None] * 2
    out = [None] * 2

    pe_fetch[0] = fetch_pe(0)
    x_fetch[0] = fetch_x(0)
    for j in range(_NSUB):
        sc, b = j // _B, j % _B
        nxt = j + 1
        if nxt < _NSUB:
            slot = nxt % 2
            if out[slot] is not None:
                out[slot].wait()
                out[slot] = None
            x_fetch[slot] = fetch_x(nxt)
        if b == 0 and sc + 1 < _NSC:
            pe_fetch[(sc + 1) % 2] = fetch_pe(sc + 1)
        slot = j % 2
        x_fetch[slot].wait()
        if b == 0:
            pe_fetch[sc % 2].wait()
        add_loop(xbufs[slot], pbufs[sc % 2])
        out[slot] = pltpu.async_copy(
            xbufs[slot], out_hbm.at[pl.ds(x_off(j), _CHUNK)], out_sems[slot])
    for d in out:
        if d is not None:
            d.wait()


def kernel(x, pe_table):
    B, S, Dm = x.shape
    out = _sc_add(x.reshape(-1), pe_table.reshape(-1))
    return out.reshape(B, S, Dm)


# speedup vs baseline: 4.3165x; 4.3165x over previous
import jax
import jax.numpy as jnp
from jax.experimental import pallas as pl

_S_BLK = 256


def _add_pe_block(x_ref, pe_ref, o_ref):
    o_ref[...] = x_ref[...] + pe_ref[...][None, :, :]


def kernel(x, pe_table):
    B, S, D = x.shape
    pe = pe_table[:S]
    return pl.pallas_call(
        _add_pe_block,
        grid=(S // _S_BLK,),
        in_specs=[
            pl.BlockSpec((B, _S_BLK, D), lambda i: (0, i, 0)),
            pl.BlockSpec((_S_BLK, D), lambda i: (i, 0)),
        ],
        out_specs=pl.BlockSpec((B, _S_BLK, D), lambda i: (0, i, 0)),
        out_shape=jax.ShapeDtypeStruct((B, S, D), x.dtype),
    )(x, pe)


# speedup vs baseline: 4.3422x; 1.0059x over previous
import jax
import jax.numpy as jnp
from jax.experimental import pallas as pl

_S_BLK = 512


def _add_pe_block(x_ref, pe_ref, o_ref):
    o_ref[...] = x_ref[...] + pe_ref[...][None, :, :]


def kernel(x, pe_table):
    B, S, D = x.shape
    pe = pe_table[:S]
    return pl.pallas_call(
        _add_pe_block,
        grid=(S // _S_BLK,),
        in_specs=[
            pl.BlockSpec((B, _S_BLK, D), lambda i: (0, i, 0)),
            pl.BlockSpec((_S_BLK, D), lambda i: (i, 0)),
        ],
        out_specs=pl.BlockSpec((B, _S_BLK, D), lambda i: (0, i, 0)),
        out_shape=jax.ShapeDtypeStruct((B, S, D), x.dtype),
    )(x, pe)
